# baseline (speedup 1.0000x reference)
import functools

import jax
import jax.numpy as jnp
from jax import lax
from jax.experimental import pallas as pl
from jax.experimental.pallas import tpu as pltpu
from jax.experimental.pallas import tpu_sc as plsc

N = 10000
E = 160000
RREL = 3
DIN = 128
D = 64

NC, NS = 2, 16
NW = NC * NS
CH = 128
TCH = E // CH
CPW = TCH // NW
NBUF = 8
NI = (CPW // NBUF) * NBUF
NPAD = 10240
RPT = N // NS
DPT = NPAD // NS


BR = 2048
GRID = NPAD // BR

N2 = N // 2
N2P = NPAD // 2
BRH = BR // 2


def _worker_span(cid, sid):
    wid = sid * NC + cid
    cstart = CPW * wid + jnp.maximum(wid - (NW - 2), 0)
    nch = CPW + (wid >= NW - 2).astype(jnp.int32)
    return cstart, nch


def _deg_body(idx_hbm, out_hbm, idxb, ones_v, zb, sem,
              a0, a1, a2, a3, a4, a5):
    accs = (a0, a1, a2, a3, a4, a5)
    cid = lax.axis_index("c")
    sid = lax.axis_index("s")
    cstart, nch = _worker_span(cid, sid)

    def fill(i, _):
        ones_v[pl.ds(i * 16, 16)] = jnp.ones((16,), jnp.float32)
        return 0
    lax.fori_loop(0, CH // 16, fill, 0)

    def fillz(i, _):
        zb[pl.ds(i * 16, 16)] = jnp.zeros((16,), jnp.float32)
        return 0
    lax.fori_loop(0, DPT // 16, fillz, 0)

    for acc in accs:
        pltpu.sync_copy(zb, acc.at[pl.ds(sid * DPT, DPT)])
    plsc.subcore_barrier()

    for k in range(6):
        acc = accs[k]
        pltpu.sync_copy(idx_hbm.at[k, pl.ds(cstart, CPW + 1)], idxb)

        def perm(j, _):
            for t in range(CH // 16):
                v = idxb[j, pl.ds(t * 16, 16)]
                idxb[j, pl.ds(t * 16, 16)] = (
                    (v & 1) * N2P + lax.shift_right_logical(v, 1))
            return 0
        lax.fori_loop(0, CPW + 1, perm, 0)

        def fire(j, _):
            pltpu.make_async_copy(ones_v, acc.at[idxb.at[j]], sem).start(add=True)
            return 0
        lax.fori_loop(0, nch, fire, 0)

        def drain(j, _):
            pltpu.make_async_copy(ones_v, acc.at[idxb.at[0]], sem).wait()
            return 0
        lax.fori_loop(0, nch, drain, 0)

    plsc.subcore_barrier()
    for k in range(6):
        pltpu.sync_copy(accs[k].at[pl.ds(sid * DPT, DPT)],
                        out_hbm.at[cid, k, pl.ds(sid * DPT, DPT)])


_DEG_SCRATCH = [
    pltpu.VMEM((CPW + 1, CH), jnp.int32),
    pltpu.VMEM((CH,), jnp.float32),
    pltpu.VMEM((DPT,), jnp.float32),
    pltpu.SemaphoreType.DMA,
] + [pltpu.VMEM_SHARED((NPAD,), jnp.float32) for _ in range(6)]

@functools.cache
def _sc_calls():
    mesh = plsc.VectorSubcoreMesh(core_axis_name="c", subcore_axis_name="s",
                                  num_cores=NC, num_subcores=NS)
    params = pltpu.CompilerParams(use_tc_tiling_on_sc=False)
    sc_deg = pl.kernel(
        _deg_body,
        out_type=jax.ShapeDtypeStruct((NC, 6, NPAD), jnp.float32),
        mesh=mesh,
        scratch_types=_DEG_SCRATCH,
        compiler_params=params,
    )
    sc_aggs = [
        pl.kernel(
            _make_agg_body(r),
            out_type=jax.ShapeDtypeStruct((NC, N, D), jnp.bfloat16),
            mesh=mesh,
            scratch_types=_AGG_SCRATCH,
            compiler_params=params,
        )
        for r in range(RREL)
    ]
    return sc_deg, sc_aggs


ZBR = 125


def _make_agg_body(r):
    def _agg_body(z_hbm, idx_hbm, out_hbm,
                  idxs, idxd,
                  r0, r1, r2, r3, r4, r5, r6, r7, zb,
                  acc,
                  sg0, sg1, sg2, sg3, sg4, sg5, sg6, sg7,
                  ss0, ss1, ss2, ss3, ss4, ss5, ss6, ss7):
        rows = (r0, r1, r2, r3, r4, r5, r6, r7)
        semg = (sg0, sg1, sg2, sg3, sg4, sg5, sg6, sg7)
        sems = (ss0, ss1, ss2, ss3, ss4, ss5, ss6, ss7)
        cid = lax.axis_index("c")
        sid = lax.axis_index("s")
        cstart, nch = _worker_span(cid, sid)

        def zrow(i, _):
            for t in range(D // 32):
                zb[i, pl.ds(t * 32, 32)] = jnp.zeros((32,), jnp.bfloat16)
            return 0
        lax.fori_loop(0, ZBR, zrow, 0)
        for t in range(RPT // ZBR):
            pltpu.sync_copy(zb, acc.at[pl.ds(sid * RPT + t * ZBR, ZBR)])
        plsc.subcore_barrier()

        z = z_hbm
        pltpu.sync_copy(idx_hbm.at[2 * r, pl.ds(cstart, CPW + 1)], idxs)
        pltpu.sync_copy(idx_hbm.at[2 * r + 1, pl.ds(cstart, CPW + 1)], idxd)

        def gather(j, b):
            pltpu.make_async_copy(z.at[idxs.at[j]], rows[b], semg[b]).start()

        def gather_wait(j, b):
            pltpu.make_async_copy(z.at[idxs.at[j]], rows[b], semg[b]).wait()

        def scat(j, b):
            d = pltpu.make_async_copy(rows[b], acc.at[idxd.at[j]], sems[b])
            d.start(add=True)
            pltpu.make_async_copy(rows[b], acc.at[idxd.at[j]], sems[b]).wait()

        for b in range(NBUF):
            gather(b, b)

        def ring(i, _):
            for b in range(NBUF):
                j = i * NBUF + b
                gather_wait(j, b)
                scat(j, b)
                jn = j + NBUF

                @pl.when(jn < nch)
                def _():
                    gather(jn, b)
            return 0
        lax.fori_loop(0, NI // NBUF, ring, 0)

        for b in range(CPW - NI):
            gather_wait(NI + b, b)
            scat(NI + b, b)

        @pl.when(nch > CPW)
        def _():
            b = CPW - NI
            gather_wait(CPW, b)
            scat(CPW, b)

        plsc.subcore_barrier()
        pltpu.sync_copy(acc.at[pl.ds(sid * RPT, RPT)],
                        out_hbm.at[cid, pl.ds(sid * RPT, RPT)])
    return _agg_body


_AGG_SCRATCH = [
    pltpu.VMEM((CPW + 1, CH), jnp.int32),
    pltpu.VMEM((CPW + 1, CH), jnp.int32),
] + [pltpu.VMEM((CH, D), jnp.bfloat16) for _ in range(NBUF)] + [
    pltpu.VMEM((ZBR, D), jnp.bfloat16),
    pltpu.VMEM_SHARED((N, D), jnp.bfloat16),
] + [pltpu.SemaphoreType.DMA for _ in range(2 * NBUF)]


def _mm_body(feat_ref, w0_ref, y_ref):
    y_ref[...] = jnp.dot(feat_ref[...], w0_ref[...],
                         preferred_element_type=jnp.float32)


def _d1_body(deg_ref, sc_ref):
    deg = jnp.maximum(deg_ref[0] + deg_ref[1], 1.0)
    s = lax.rsqrt(deg)
    for r in range(RREL):
        sc_ref[0, r] = s[2 * r]
        sc_ref[1, r] = s[2 * r + 1]


def _make_zscale_body(r):
    def body(y_ref, sce_ref, sco_ref, z_ref):
        y = y_ref[...]
        z_ref[...] = jnp.concatenate(
            [y[:, :D] * sce_ref[0, r][:, None],
             y[:, D:] * sco_ref[0, r][:, None]], axis=1).astype(jnp.bfloat16)
    return body


def _ln_elu(agg, si, b, g, be):
    x = agg * si[..., None] + b
    h = jnp.where(x > 0, x, jnp.exp(jnp.minimum(x, 0.0)) - 1.0)
    mu = jnp.mean(h, axis=-1, keepdims=True)
    var = jnp.mean((h - mu) * (h - mu), axis=-1, keepdims=True)
    return (h - mu) * lax.rsqrt(var + 1e-5) * g + be


def _halved_agg(a_ref):
    ae = a_ref[0, :, :D].astype(jnp.float32) + a_ref[1, :, :D].astype(jnp.float32)
    ao = a_ref[0, :, D:].astype(jnp.float32) + a_ref[1, :, D:].astype(jnp.float32)
    return ae, ao


def _make_ln0_body(r):
    def hn_pair(a_ref, sce_ref, sco_ref, b_ref, g_ref, be_ref):
        ae, ao = _halved_agg(a_ref)
        hne = _ln_elu(ae, sce_ref[1, r], b_ref[...], g_ref[...], be_ref[...])
        hno = _ln_elu(ao, sco_ref[1, r], b_ref[...], g_ref[...], be_ref[...])
        return jnp.concatenate([hne, hno], axis=1)

    if r == 0:
        def body(a_ref, sce_ref, sco_ref, b_ref, g_ref, be_ref, h1_ref):
            h1_ref[...] = hn_pair(a_ref, sce_ref, sco_ref, b_ref, g_ref, be_ref)
    else:
        def body(a_ref, sce_ref, sco_ref, b_ref, g_ref, be_ref, h1in_ref,
                 h1_ref):
            h1_ref[...] = h1in_ref[...] + hn_pair(a_ref, sce_ref, sco_ref,
                                                  b_ref, g_ref, be_ref)
    return body


def _make_ln1_body(r):
    def body(a_ref, sce_ref, sco_ref, b_ref, g_ref, be_ref, ws1_ref, ws2_ref,
             hn_ref, lge_ref, lgo_ref):
        ae, ao = _halved_agg(a_ref)
        hne = _ln_elu(ae, sce_ref[1, r], b_ref[...], g_ref[...], be_ref[...])
        hno = _ln_elu(ao, sco_ref[1, r], b_ref[...], g_ref[...], be_ref[...])
        hn_ref[...] = jnp.concatenate([hne, hno], axis=1)
        w2 = ws2_ref[r][None, :]
        te = jax.nn.sigmoid(jnp.dot(hne, ws1_ref[r],
                                    preferred_element_type=jnp.float32))
        to = jax.nn.sigmoid(jnp.dot(hno, ws1_ref[r],
                                    preferred_element_type=jnp.float32))
        lge_ref[...] = jnp.sum(te * w2, axis=-1).reshape(1, -1)
        lgo_ref[...] = jnp.sum(to * w2, axis=-1).reshape(1, -1)
    return body


def _d2c_body(h1_ref, sce_ref, sco_ref, w1_ref, z0_ref, z1_ref, z2_ref):
    h1 = h1_ref[...]
    ye = jnp.dot(h1[:, :D], w1_ref[...], preferred_element_type=jnp.float32)
    yo = jnp.dot(h1[:, D:], w1_ref[...], preferred_element_type=jnp.float32)
    for r, z_ref in enumerate((z0_ref, z1_ref, z2_ref)):
        z_ref[...] = jnp.concatenate(
            [ye * sce_ref[0, r][:, None],
             yo * sco_ref[0, r][:, None]], axis=1).astype(jnp.bfloat16)


def _softmax3(lg):
    m = jnp.max(lg, axis=0, keepdims=True)
    e = jnp.exp(lg - m)
    return e / jnp.sum(e, axis=0, keepdims=True)


def _d3c_body(hn0_ref, hn1_ref, hn2_ref, lge_ref, lgo_ref, h1_ref, al_ref,
              out_ref, atte_ref, atto_ref):
    hns = (hn0_ref[...], hn1_ref[...], hn2_ref[...])
    atte = _softmax3(lge_ref[...])
    atto = _softmax3(lgo_ref[...])
    h2e = sum(atte[r][:, None] * hns[r][:, :D] for r in range(RREL))
    h2o = sum(atto[r][:, None] * hns[r][:, D:] for r in range(RREL))
    h2 = jnp.concatenate([h2e, h2o], axis=1)
    a = jax.nn.sigmoid(al_ref[0, 0])
    out_ref[...] = a * h2 + (1.0 - a) * h1_ref[...]
    atte_ref[...] = atte
    atto_ref[...] = atto


_PAIR_SPEC = pl.BlockSpec((BRH, 2 * D), lambda i: (i, 0))
_SCE_SPEC = pl.BlockSpec((2, RREL, BRH), lambda i: (0, 0, i))
_SCO_SPEC = pl.BlockSpec((2, RREL, BRH), lambda i: (0, 0, i + NPAD // (2 * BRH)))
_APAIR_SPEC = pl.BlockSpec((NC, BRH, 2 * D), lambda i: (0, i, 0))
_LG_SPEC = pl.BlockSpec((1, BRH), lambda i: (0, i))
_LG3_SPEC = pl.BlockSpec((RREL, BRH), lambda i: (0, i))


def _dense1a(feat_p, w2):
    return pl.pallas_call(
        _mm_body,
        grid=(GRID,),
        in_specs=[
            pl.BlockSpec((BRH, 2 * DIN), lambda i: (i, 0)),
            pl.BlockSpec((2 * DIN, 2 * D), lambda i: (0, 0)),
        ],
        out_specs=_PAIR_SPEC,
        out_shape=jax.ShapeDtypeStruct((N2P, 2 * D), jnp.float32),
    )(feat_p, w2)


def _dense1b(deg):
    return pl.pallas_call(
        _d1_body,
        grid=(GRID,),
        in_specs=[
            pl.BlockSpec((NC, 6, BR), lambda i: (0, 0, i)),
        ],
        out_specs=pl.BlockSpec((2, RREL, BR), lambda i: (0, 0, i)),
        out_shape=jax.ShapeDtypeStruct((2, RREL, NPAD), jnp.float32),
    )(deg)


def _zscale(r, y, scales):
    return pl.pallas_call(
        _make_zscale_body(r),
        grid=(GRID,),
        in_specs=[_PAIR_SPEC, _SCE_SPEC, _SCO_SPEC],
        out_specs=_PAIR_SPEC,
        out_shape=jax.ShapeDtypeStruct((N2P, 2 * D), jnp.bfloat16),
    )(y, scales, scales)


_VEC_SPEC = pl.BlockSpec((1, D), lambda i: (0, 0))


def _ln0(r, agg, scales, b, g, be, h1=None):
    in_specs = [_APAIR_SPEC, _SCE_SPEC, _SCO_SPEC,
                _VEC_SPEC, _VEC_SPEC, _VEC_SPEC]
    args = [agg, scales, scales, b, g, be]
    if r > 0:
        in_specs.append(_PAIR_SPEC)
        args.append(h1)
    return pl.pallas_call(
        _make_ln0_body(r),
        grid=(GRID,),
        in_specs=in_specs,
        out_specs=_PAIR_SPEC,
        out_shape=jax.ShapeDtypeStruct((N2P, 2 * D), jnp.float32),
    )(*args)


def _ln1(r, agg, scales, b, g, be, ws1, ws2):
    return pl.pallas_call(
        _make_ln1_body(r),
        grid=(GRID,),
        in_specs=[
            _APAIR_SPEC, _SCE_SPEC, _SCO_SPEC,
            _VEC_SPEC, _VEC_SPEC, _VEC_SPEC,
            pl.BlockSpec((RREL, D, D), lambda i: (0, 0, 0)),
            pl.BlockSpec((RREL, D), lambda i: (0, 0)),
        ],
        out_specs=[_PAIR_SPEC, _LG_SPEC, _LG_SPEC],
        out_shape=[
            jax.ShapeDtypeStruct((N2P, 2 * D), jnp.float32),
            jax.ShapeDtypeStruct((1, N2P), jnp.float32),
            jax.ShapeDtypeStruct((1, N2P), jnp.float32),
        ],
    )(agg, scales, scales, b, g, be, ws1, ws2)


def _dense2c(h1, scales, w1):
    return pl.pallas_call(
        _d2c_body,
        grid=(GRID,),
        in_specs=[
            _PAIR_SPEC, _SCE_SPEC, _SCO_SPEC,
            pl.BlockSpec((D, D), lambda i: (0, 0)),
        ],
        out_specs=[_PAIR_SPEC] * RREL,
        out_shape=[jax.ShapeDtypeStruct((N2P, 2 * D), jnp.bfloat16)] * RREL,
    )(h1, scales, scales, w1)


def _dense3c(hn0, hn1, hn2, lge, lgo, h1, alpha):
    return pl.pallas_call(
        _d3c_body,
        grid=(GRID,),
        in_specs=[
            _PAIR_SPEC, _PAIR_SPEC, _PAIR_SPEC,
            _LG3_SPEC, _LG3_SPEC,
            _PAIR_SPEC,
            pl.BlockSpec((1, 1), lambda i: (0, 0)),
        ],
        out_specs=[_PAIR_SPEC, _LG3_SPEC, _LG3_SPEC],
        out_shape=[
            jax.ShapeDtypeStruct((N2P, 2 * D), jnp.float32),
            jax.ShapeDtypeStruct((RREL, N2P), jnp.float32),
            jax.ShapeDtypeStruct((RREL, N2P), jnp.float32),
        ],
    )(hn0, hn1, hn2, lge, lgo, h1, alpha)


def kernel(feat, edge_index_r0, edge_index_r1, edge_index_r2,
           W0, b0, g0, be0, Ws1_0, Ws2_0,
           W1, b1, g1, be1, Ws1_1, Ws2_1, alpha):
    ei = jnp.stack([edge_index_r0, edge_index_r1, edge_index_r2]).astype(jnp.int32)
    idx6 = ei.reshape(6, TCH, CH)
    feat_p = feat.reshape(N2, 2 * DIN)
    w2 = jnp.zeros((2 * DIN, 2 * D), W0.dtype)
    w2 = w2.at[:DIN, :D].set(W0).at[DIN:, D:].set(W0)

    sc_deg, sc_aggs = _sc_calls()
    deg = sc_deg(idx6)
    y0 = _dense1a(feat_p, w2)

    scales = _dense1b(deg)
    b0r, g0r, be0r = b0.reshape(1, D), g0.reshape(1, D), be0.reshape(1, D)

    z0s = [_zscale(r, y0, scales) for r in range(RREL)]
    agg0 = [sc_aggs[r](z0s[r].reshape(NPAD, D), idx6) for r in range(RREL)]
    h1 = None
    for r in range(RREL):
        h1 = _ln0(r, agg0[r].reshape(NC, N2, 2 * D), scales,
                  b0r, g0r, be0r, h1)

    z1s = _dense2c(h1, scales, W1)

    b1r, g1r, be1r = b1.reshape(1, D), g1.reshape(1, D), be1.reshape(1, D)
    ws2 = Ws2_1[:, :, 0]
    agg1 = [sc_aggs[r](z1s[r].reshape(NPAD, D), idx6) for r in range(RREL)]
    hns, lges, lgos = [], [], []
    for r in range(RREL):
        hn_r, lge_r, lgo_r = _ln1(r, agg1[r].reshape(NC, N2, 2 * D), scales,
                                  b1r, g1r, be1r, Ws1_1, ws2)
        hns.append(hn_r)
        lges.append(lge_r)
        lgos.append(lgo_r)
    lge = jnp.concatenate(lges, axis=0)
    lgo = jnp.concatenate(lgos, axis=0)
    h_pair, atte, atto = _dense3c(hns[0], hns[1], hns[2], lge, lgo, h1,
                                  alpha.reshape(1, 1))
    h = h_pair.reshape(NPAD, D)[:N]
    att = jnp.stack([atte[:, :N2], atto[:, :N2]], axis=2).reshape(RREL, N)
    return h, att.T


# speedup vs baseline: 1.0331x; 1.0331x over previous
import functools

import jax
import jax.numpy as jnp
from jax import lax
from jax.experimental import pallas as pl
from jax.experimental.pallas import tpu as pltpu
from jax.experimental.pallas import tpu_sc as plsc

N = 10000
E = 160000
RREL = 3
DIN = 128
D = 64

NC, NS = 2, 16
NW = NC * NS
CH = 128
TCH = E // CH
CPW = TCH // NW
NBUF = 8
NI = (CPW // NBUF) * NBUF
NPAD = 10240
RPT = N // NS
DPT = NPAD // NS


BR = 2048
GRID = NPAD // BR


def _worker_span(cid, sid):
    wid = sid * NC + cid
    cstart = CPW * wid + jnp.maximum(wid - (NW - 2), 0)
    nch = CPW + (wid >= NW - 2).astype(jnp.int32)
    return cstart, nch


def _deg_body(idx_hbm, out_hbm, idxb, ones_v, zb, sem,
              a0, a1, a2, a3, a4, a5):
    accs = (a0, a1, a2, a3, a4, a5)
    cid = lax.axis_index("c")
    sid = lax.axis_index("s")
    cstart, nch = _worker_span(cid, sid)

    def fill(i, _):
        ones_v[pl.ds(i * 16, 16)] = jnp.ones((16,), jnp.float32)
        return 0
    lax.fori_loop(0, CH // 16, fill, 0)

    def fillz(i, _):
        zb[pl.ds(i * 16, 16)] = jnp.zeros((16,), jnp.float32)
        return 0
    lax.fori_loop(0, DPT // 16, fillz, 0)

    for acc in accs:
        pltpu.sync_copy(zb, acc.at[pl.ds(sid * DPT, DPT)])
    plsc.subcore_barrier()

    for k in range(6):
        acc = accs[k]
        pltpu.sync_copy(idx_hbm.at[k, pl.ds(cstart, CPW + 1)], idxb)

        def fire(j, _):
            pltpu.make_async_copy(ones_v, acc.at[idxb.at[j]], sem).start(add=True)
            return 0
        lax.fori_loop(0, nch, fire, 0)

        def drain(j, _):
            pltpu.make_async_copy(ones_v, acc.at[idxb.at[0]], sem).wait()
            return 0
        lax.fori_loop(0, nch, drain, 0)

    plsc.subcore_barrier()
    for k in range(6):
        pltpu.sync_copy(accs[k].at[pl.ds(sid * DPT, DPT)],
                        out_hbm.at[cid, k, pl.ds(sid * DPT, DPT)])


_DEG_SCRATCH = [
    pltpu.VMEM((CPW + 1, CH), jnp.int32),
    pltpu.VMEM((CH,), jnp.float32),
    pltpu.VMEM((DPT,), jnp.float32),
    pltpu.SemaphoreType.DMA,
] + [pltpu.VMEM_SHARED((NPAD,), jnp.float32) for _ in range(6)]

@functools.cache
def _sc_calls():
    mesh = plsc.VectorSubcoreMesh(core_axis_name="c", subcore_axis_name="s",
                                  num_cores=NC, num_subcores=NS)
    params = pltpu.CompilerParams(use_tc_tiling_on_sc=False)
    sc_deg = pl.kernel(
        _deg_body,
        out_type=jax.ShapeDtypeStruct((NC, 6, NPAD), jnp.float32),
        mesh=mesh,
        scratch_types=_DEG_SCRATCH,
        compiler_params=params,
    )
    sc_aggs = [
        pl.kernel(
            _make_agg_body(r),
            out_type=jax.ShapeDtypeStruct((NC, N, D), jnp.bfloat16),
            mesh=mesh,
            scratch_types=_AGG_SCRATCH,
            compiler_params=params,
        )
        for r in range(RREL)
    ]
    return sc_deg, sc_aggs


ZBR = 125


def _make_agg_body(r):
    def _agg_body(z_hbm, idx_hbm, out_hbm,
                  idxs, idxd,
                  r0, r1, r2, r3, r4, r5, r6, r7, zb,
                  acc,
                  sg0, sg1, sg2, sg3, sg4, sg5, sg6, sg7,
                  ss0, ss1, ss2, ss3, ss4, ss5, ss6, ss7):
        rows = (r0, r1, r2, r3, r4, r5, r6, r7)
        semg = (sg0, sg1, sg2, sg3, sg4, sg5, sg6, sg7)
        sems = (ss0, ss1, ss2, ss3, ss4, ss5, ss6, ss7)
        cid = lax.axis_index("c")
        sid = lax.axis_index("s")
        cstart, nch = _worker_span(cid, sid)

        def zrow(i, _):
            for t in range(D // 32):
                zb[i, pl.ds(t * 32, 32)] = jnp.zeros((32,), jnp.bfloat16)
            return 0
        lax.fori_loop(0, ZBR, zrow, 0)
        for t in range(RPT // ZBR):
            pltpu.sync_copy(zb, acc.at[pl.ds(sid * RPT + t * ZBR, ZBR)])
        plsc.subcore_barrier()

        z = z_hbm
        pltpu.sync_copy(idx_hbm.at[2 * r, pl.ds(cstart, CPW + 1)], idxs)
        pltpu.sync_copy(idx_hbm.at[2 * r + 1, pl.ds(cstart, CPW + 1)], idxd)

        def gather(j, b):
            pltpu.make_async_copy(z.at[idxs.at[j]], rows[b], semg[b]).start()

        def gather_wait(j, b):
            pltpu.make_async_copy(z.at[idxs.at[j]], rows[b], semg[b]).wait()

        def scat(j, b):
            d = pltpu.make_async_copy(rows[b], acc.at[idxd.at[j]], sems[b])
            d.start(add=True)
            pltpu.make_async_copy(rows[b], acc.at[idxd.at[j]], sems[b]).wait()

        for b in range(NBUF):
            gather(b, b)

        def ring(i, _):
            for b in range(NBUF):
                j = i * NBUF + b
                gather_wait(j, b)
                scat(j, b)
                jn = j + NBUF

                @pl.when(jn < nch)
                def _():
                    gather(jn, b)
            return 0
        lax.fori_loop(0, NI // NBUF, ring, 0)

        for b in range(CPW - NI):
            gather_wait(NI + b, b)
            scat(NI + b, b)

        @pl.when(nch > CPW)
        def _():
            b = CPW - NI
            gather_wait(CPW, b)
            scat(CPW, b)

        plsc.subcore_barrier()
        pltpu.sync_copy(acc.at[pl.ds(sid * RPT, RPT)],
                        out_hbm.at[cid, pl.ds(sid * RPT, RPT)])
    return _agg_body


_AGG_SCRATCH = [
    pltpu.VMEM((CPW + 1, CH), jnp.int32),
    pltpu.VMEM((CPW + 1, CH), jnp.int32),
] + [pltpu.VMEM((CH, D), jnp.bfloat16) for _ in range(NBUF)] + [
    pltpu.VMEM((ZBR, D), jnp.bfloat16),
    pltpu.VMEM_SHARED((N, D), jnp.bfloat16),
] + [pltpu.SemaphoreType.DMA for _ in range(2 * NBUF)]


def _mm_body(feat_ref, w0_ref, y_ref):
    y_ref[...] = jnp.dot(feat_ref[...], w0_ref[...],
                         preferred_element_type=jnp.float32)


def _d1_body(deg_ref, sc_ref):
    deg = jnp.maximum(deg_ref[0] + deg_ref[1], 1.0)
    s = lax.rsqrt(deg)
    for r in range(RREL):
        sc_ref[0, r] = s[2 * r]
        sc_ref[1, r] = s[2 * r + 1]


def _make_zscale_body(r):
    def body(y_ref, sc_ref, z_ref):
        z_ref[...] = (y_ref[...] * sc_ref[0, r][:, None]).astype(jnp.bfloat16)
    return body


def _ln_elu(agg, si, b, g, be):
    x = agg * si[..., None] + b
    h = jnp.where(x > 0, x, jnp.exp(jnp.minimum(x, 0.0)) - 1.0)
    mu = jnp.mean(h, axis=-1, keepdims=True)
    var = jnp.mean((h - mu) * (h - mu), axis=-1, keepdims=True)
    return (h - mu) * lax.rsqrt(var + 1e-5) * g + be


def _make_ln0_body(r):
    if r == 0:
        def body(a_ref, sc_ref, b_ref, g_ref, be_ref, h1_ref):
            agg = a_ref[0].astype(jnp.float32) + a_ref[1].astype(jnp.float32)
            h1_ref[...] = _ln_elu(agg, sc_ref[1, r], b_ref[...], g_ref[...],
                                  be_ref[...])
    else:
        def body(a_ref, sc_ref, b_ref, g_ref, be_ref, h1in_ref, h1_ref):
            agg = a_ref[0].astype(jnp.float32) + a_ref[1].astype(jnp.float32)
            h1_ref[...] = h1in_ref[...] + _ln_elu(
                agg, sc_ref[1, r], b_ref[...], g_ref[...], be_ref[...])
    return body


def _make_ln1_body(r):
    def body(a_ref, sc_ref, b_ref, g_ref, be_ref, ws1_ref, ws2_ref,
             hn_ref, lg_ref):
        agg = a_ref[0].astype(jnp.float32) + a_ref[1].astype(jnp.float32)
        hn = _ln_elu(agg, sc_ref[1, r], b_ref[...], g_ref[...], be_ref[...])
        t = jax.nn.sigmoid(jnp.dot(hn, ws1_ref[r],
                                   preferred_element_type=jnp.float32))
        hn_ref[...] = hn
        lg_ref[...] = jnp.sum(t * ws2_ref[r][None, :], axis=-1).reshape(1, -1)
    return body


def _d2c_body(h1_ref, sc_ref, w1_ref, z0_ref, z1_ref, z2_ref):
    y1 = jnp.dot(h1_ref[...], w1_ref[...], preferred_element_type=jnp.float32)
    for r, z_ref in enumerate((z0_ref, z1_ref, z2_ref)):
        z_ref[...] = (y1 * sc_ref[0, r][:, None]).astype(jnp.bfloat16)


def _d3c_body(hn0_ref, hn1_ref, hn2_ref, lg_ref, h1_ref, al_ref,
              out_ref, att_ref):
    hns = (hn0_ref[...], hn1_ref[...], hn2_ref[...])
    lg = lg_ref[...]
    m = jnp.max(lg, axis=0, keepdims=True)
    e = jnp.exp(lg - m)
    att = e / jnp.sum(e, axis=0, keepdims=True)
    h2 = (att[0][:, None] * hns[0] + att[1][:, None] * hns[1]
          + att[2][:, None] * hns[2])
    a = jax.nn.sigmoid(al_ref[0, 0])
    out_ref[...] = a * h2 + (1.0 - a) * h1_ref[...]
    att_ref[...] = att


def _dense1a(feat, w0):
    return pl.pallas_call(
        _mm_body,
        grid=(GRID,),
        in_specs=[
            pl.BlockSpec((BR, DIN), lambda i: (i, 0)),
            pl.BlockSpec((DIN, D), lambda i: (0, 0)),
        ],
        out_specs=pl.BlockSpec((BR, D), lambda i: (i, 0)),
        out_shape=jax.ShapeDtypeStruct((NPAD, D), jnp.float32),
    )(feat, w0)


def _dense1b(deg):
    return pl.pallas_call(
        _d1_body,
        grid=(GRID,),
        in_specs=[
            pl.BlockSpec((NC, 6, BR), lambda i: (0, 0, i)),
        ],
        out_specs=pl.BlockSpec((2, RREL, BR), lambda i: (0, 0, i)),
        out_shape=jax.ShapeDtypeStruct((2, RREL, NPAD), jnp.float32),
    )(deg)


def _zscale(r, y, scales):
    return pl.pallas_call(
        _make_zscale_body(r),
        grid=(GRID,),
        in_specs=[
            pl.BlockSpec((BR, D), lambda i: (i, 0)),
            pl.BlockSpec((2, RREL, BR), lambda i: (0, 0, i)),
        ],
        out_specs=pl.BlockSpec((BR, D), lambda i: (i, 0)),
        out_shape=jax.ShapeDtypeStruct((NPAD, D), jnp.bfloat16),
    )(y, scales)


_AGG_SPEC = pl.BlockSpec((NC, BR, D), lambda i: (0, i, 0))
_SC_SPEC = pl.BlockSpec((2, RREL, BR), lambda i: (0, 0, i))
_VEC_SPEC = pl.BlockSpec((1, D), lambda i: (0, 0))
_ROW_SPEC = pl.BlockSpec((BR, D), lambda i: (i, 0))


def _ln0(r, agg, scales, b, g, be, h1=None):
    in_specs = [_AGG_SPEC, _SC_SPEC, _VEC_SPEC, _VEC_SPEC, _VEC_SPEC]
    args = [agg, scales, b, g, be]
    if r > 0:
        in_specs.append(_ROW_SPEC)
        args.append(h1)
    return pl.pallas_call(
        _make_ln0_body(r),
        grid=(GRID,),
        in_specs=in_specs,
        out_specs=_ROW_SPEC,
        out_shape=jax.ShapeDtypeStruct((NPAD, D), jnp.float32),
    )(*args)


def _ln1(r, agg, scales, b, g, be, ws1, ws2):
    return pl.pallas_call(
        _make_ln1_body(r),
        grid=(GRID,),
        in_specs=[
            _AGG_SPEC, _SC_SPEC, _VEC_SPEC, _VEC_SPEC, _VEC_SPEC,
            pl.BlockSpec((RREL, D, D), lambda i: (0, 0, 0)),
            pl.BlockSpec((RREL, D), lambda i: (0, 0)),
        ],
        out_specs=[
            _ROW_SPEC,
            pl.BlockSpec((1, BR), lambda i: (0, i)),
        ],
        out_shape=[
            jax.ShapeDtypeStruct((NPAD, D), jnp.float32),
            jax.ShapeDtypeStruct((1, NPAD), jnp.float32),
        ],
    )(agg, scales, b, g, be, ws1, ws2)


def _dense2c(h1, scales, w1):
    return pl.pallas_call(
        _d2c_body,
        grid=(GRID,),
        in_specs=[
            _ROW_SPEC,
            _SC_SPEC,
            pl.BlockSpec((D, D), lambda i: (0, 0)),
        ],
        out_specs=[pl.BlockSpec((BR, D), lambda i: (i, 0))] * RREL,
        out_shape=[jax.ShapeDtypeStruct((NPAD, D), jnp.bfloat16)] * RREL,
    )(h1, scales, w1)


def _dense3c(hn0, hn1, hn2, lgs, h1, alpha):
    return pl.pallas_call(
        _d3c_body,
        grid=(GRID,),
        in_specs=[
            _ROW_SPEC, _ROW_SPEC, _ROW_SPEC,
            pl.BlockSpec((RREL, BR), lambda i: (0, i)),
            _ROW_SPEC,
            pl.BlockSpec((1, 1), lambda i: (0, 0)),
        ],
        out_specs=[
            pl.BlockSpec((BR, D), lambda i: (i, 0)),
            pl.BlockSpec((RREL, BR), lambda i: (0, i)),
        ],
        out_shape=[
            jax.ShapeDtypeStruct((N, D), jnp.float32),
            jax.ShapeDtypeStruct((RREL, N), jnp.float32),
        ],
    )(hn0, hn1, hn2, lgs, h1, alpha)


def kernel(feat, edge_index_r0, edge_index_r1, edge_index_r2,
           W0, b0, g0, be0, Ws1_0, Ws2_0,
           W1, b1, g1, be1, Ws1_1, Ws2_1, alpha):
    ei = jnp.stack([edge_index_r0, edge_index_r1, edge_index_r2]).astype(jnp.int32)
    idx6 = ei.reshape(6, TCH, CH)

    sc_deg, sc_aggs = _sc_calls()
    deg = sc_deg(idx6)
    y0 = _dense1a(feat, W0)

    scales = _dense1b(deg)
    b0r, g0r, be0r = b0.reshape(1, D), g0.reshape(1, D), be0.reshape(1, D)

    z0s = [_zscale(r, y0, scales) for r in range(RREL)]
    agg0 = [sc_aggs[r](z0s[r], idx6) for r in range(RREL)]
    h1 = None
    for r in range(RREL):
        h1 = _ln0(r, agg0[r], scales, b0r, g0r, be0r, h1)

    z1s = _dense2c(h1, scales, W1)

    b1r, g1r, be1r = b1.reshape(1, D), g1.reshape(1, D), be1.reshape(1, D)
    ws2 = Ws2_1[:, :, 0]
    agg1 = [sc_aggs[r](z1s[r], idx6) for r in range(RREL)]
    hns, lgs = [], []
    for r in range(RREL):
        hn_r, lg_r = _ln1(r, agg1[r], scales, b1r, g1r, be1r, Ws1_1, ws2)
        hns.append(hn_r)
        lgs.append(lg_r)
    lg = jnp.concatenate(lgs, axis=0)
    h, att = _dense3c(hns[0], hns[1], hns[2], lg, h1, alpha.reshape(1, 1))
    return h, att.T
